# baseline (speedup 1.0000x reference)
import functools

import jax
import jax.numpy as jnp
from jax import lax
from jax.experimental import pallas as pl
from jax.experimental.pallas import tpu as pltpu
from jax.experimental.pallas import tpu_sc as plsc

_EMB_DIM = 64
_NOISE_DIM = 32
_NUM_CORES = 2
_NUM_SUBCORES = 16
_NW = _NUM_CORES * _NUM_SUBCORES

_N_ROWS = 1_000_000
_SWEEP_ROWS = (_N_ROWS // 128) * 128
_CHUNK = 512
_N_CHUNKS = _SWEEP_ROWS // _CHUNK
_CPW = _N_CHUNKS // _NW
_EXTRA = _N_CHUNKS - _CPW * _NW
_CAP = 1040
_SENT = 0x7FFFFF00


def _make_sweep_gather(batch: int):
  mesh = plsc.VectorSubcoreMesh(
      core_axis_name="c", subcore_axis_name="s",
      num_cores=_NUM_CORES, num_subcores=_NUM_SUBCORES)
  n_groups = batch // 16

  @functools.partial(
      pl.kernel,
      out_type=jax.ShapeDtypeStruct((batch, 128), jnp.float32),
      mesh=mesh,
      scratch_types=[
          pltpu.VMEM((batch + 16,), jnp.int32),
          pltpu.VMEM((_CAP + 16,), jnp.int32),
          pltpu.VMEM((_CAP + 16,), jnp.int32),
          pltpu.VMEM((_CAP + 16,), jnp.int32),
          pltpu.VMEM((_CAP + 16,), jnp.int32),
          pltpu.VMEM((2, _EMB_DIM, _CHUNK), jnp.float32),
          pltpu.VMEM((2, 16, 128), jnp.float32),
          pltpu.SemaphoreType.DMA((2, 2)),
          pltpu.SemaphoreType.DMA,
      ],
      compiler_params=pltpu.CompilerParams(needs_layout_passes=False),
  )
  def sweep_kernel(labels_hbm, tabT_hbm, out_hbm, lab_v, mylab_v, myj_v,
                   clab_v, cj_v, buf_v, row_v, sems, ssem):
    wid = lax.axis_index("s") * _NUM_CORES + lax.axis_index("c")
    n_my = _CPW + jnp.where(wid == 0, _EXTRA, 0)
    c0 = wid * _CPW + jnp.where(wid > 0, _EXTRA, 0)
    lo = c0 * _CHUNK
    hi = (c0 + n_my) * _CHUNK

    _H = _CHUNK // 2

    def _issue(t, b):
      s = pl.multiple_of((c0 + t) * _CHUNK, _CHUNK)
      for h in range(2):
        pltpu.async_copy(
            tabT_hbm.at[pl.ds(0, _EMB_DIM), pl.ds(s + h * _H, _H)],
            buf_v.at[b, pl.ds(0, _EMB_DIM), pl.ds(h * _H, _H)],
            sems.at[b, h])

    def _wait(b):
      for h in range(2):
        pltpu.make_async_copy(
            tabT_hbm.at[pl.ds(0, _EMB_DIM), pl.ds(0, _H)],
            buf_v.at[b, pl.ds(0, _EMB_DIM), pl.ds(h * _H, _H)],
            sems.at[b, h]).wait()

    _issue(0, 0)
    _issue(1, 1)

    pltpu.sync_copy(labels_hbm, lab_v.at[pl.ds(0, batch)])

    iota = lax.iota(jnp.int32, 16)

    @pl.loop(0, (_CAP + 16) // 16)
    def _fill(g):
      mylab_v[pl.ds(g * 16, 16)] = jnp.full((16,), _SENT, jnp.int32)

    @pl.loop(0, n_groups, init_carry=jnp.int32(0))
    def _compact(g, cur):
      v = lab_v[pl.ds(g * 16, 16)]
      jv = g * 16 + iota
      m = (v >= lo) & (v < hi)
      plsc.store_compressed(mylab_v.at[pl.ds(cur, 16)], v, mask=m)
      plsc.store_compressed(myj_v.at[pl.ds(cur, 16)], jv, mask=m)
      return cur + plsc.all_reduce_population_count(m)[0]

    cnt = _compact
    ng1 = lax.div(cnt + 15, jnp.int32(16))

    def _process(t, b):
      s = (c0 + t) * _CHUNK

      @pl.loop(0, (_CAP + 16) // 16)
      def _cfill(g):
        sl = pl.ds(g * 16, 16)
        clab_v[sl] = jnp.zeros((16,), jnp.int32)
        cj_v[sl] = jnp.full((16,), -1, jnp.int32)

      @pl.loop(0, ng1, init_carry=jnp.int32(0))
      def _cc(g, cur):
        v = mylab_v[pl.ds(g * 16, 16)]
        jv = myj_v[pl.ds(g * 16, 16)]
        m = (v >= s) & (v < s + _CHUNK)
        plsc.store_compressed(clab_v.at[pl.ds(cur, 16)], v - s, mask=m)
        plsc.store_compressed(cj_v.at[pl.ds(cur, 16)], jv, mask=m)
        return cur + plsc.all_reduce_population_count(m)[0]

      cnt2 = _cc
      nw = lax.div(cnt2 + 15, jnp.int32(16))

      @pl.loop(0, nw)
      def _wave(wv):
        r = wv & 1

        @pl.when(wv >= 2)
        def _():
          pltpu.make_async_copy(
              row_v.at[0],
              out_hbm.at[plsc.Indices(cj_v.at[pl.ds(0, 16)],
                                      ignored_value=-1)],
              ssem).wait()

        cols = clab_v[pl.ds(wv * 16, 16)]
        for l in range(16):
          col = cols[l]
          for k in range(_EMB_DIM // 16):
            vals = plsc.load_gather(
                buf_v.at[b],
                [k * 16 + iota, jnp.full((16,), col, jnp.int32)])
            row_v[r, l, pl.ds(k * 16, 16)] = vals

        pltpu.async_copy(
            row_v.at[r],
            out_hbm.at[plsc.Indices(cj_v.at[pl.ds(wv * 16, 16)],
                                    ignored_value=-1)],
            ssem)

      @pl.when(nw >= 1)
      def _():
        pltpu.make_async_copy(
            row_v.at[0],
            out_hbm.at[plsc.Indices(cj_v.at[pl.ds(0, 16)],
                                    ignored_value=-1)],
            ssem).wait()

      @pl.when(nw >= 2)
      def _():
        pltpu.make_async_copy(
            row_v.at[0],
            out_hbm.at[plsc.Indices(cj_v.at[pl.ds(0, 16)],
                                    ignored_value=-1)],
            ssem).wait()

    @pl.loop(0, (_CPW - 1) // 2)
    def _sweep(q):
      t0 = q * 2
      _wait(0)
      _process(t0, 0)
      _issue(t0 + 2, 0)
      _wait(1)
      _process(t0 + 1, 1)
      @pl.when(t0 + 3 <= _CPW - 1)
      def _():
        _issue(t0 + 3, 1)

    _wait(0)
    _process(_CPW - 1, 0)

    @pl.when(wid == 0)
    def _():
      _issue(_CPW, 1)
      _wait(1)
      _process(_CPW, 1)

  return sweep_kernel


def _post_kernel(embs_ref, noise_ref, lab_ref, tail_ref, out_ref, *, emb_dim):
  e = embs_ref[...][:, :emb_dim]
  lab = lab_ref[...]
  tail_id = lab - _SWEEP_ROWS
  onehot = (tail_id == jax.lax.broadcasted_iota(jnp.int32, (1, _N_ROWS - _SWEEP_ROWS), 1)).astype(jnp.float32)
  tail_e = jnp.dot(onehot, tail_ref[...], preferred_element_type=jnp.float32)
  e = jnp.where(tail_id >= 0, tail_e, e)
  mean = jnp.mean(e, axis=-1, keepdims=True)
  var = jnp.sum((e - mean) ** 2, axis=-1, keepdims=True) / (emb_dim - 1)
  std = jnp.sqrt(var)
  z = std * noise_ref[...] + mean
  out_ref[...] = jnp.concatenate((z, e), axis=-1)


def kernel(labels, table):
  batch = labels.shape[0]
  n_rows, emb_dim = table.shape
  labels = labels.astype(jnp.int32)
  noise = jax.random.normal(jax.random.key(42), (batch, _NOISE_DIM),
                            dtype=jnp.float32)

  embs_pad = _make_sweep_gather(batch)(labels, table.T)

  tail = lax.slice(table, (_SWEEP_ROWS, 0), (n_rows, emb_dim))

  blk = 2048
  out = pl.pallas_call(
      functools.partial(_post_kernel, emb_dim=emb_dim),
      grid=(batch // blk,),
      in_specs=[
          pl.BlockSpec((blk, 128), lambda i: (i, 0)),
          pl.BlockSpec((blk, _NOISE_DIM), lambda i: (i, 0)),
          pl.BlockSpec((blk, 1), lambda i: (i, 0)),
          pl.BlockSpec((n_rows - _SWEEP_ROWS, emb_dim), lambda i: (0, 0)),
      ],
      out_specs=pl.BlockSpec((blk, emb_dim + _NOISE_DIM), lambda i: (i, 0)),
      out_shape=jax.ShapeDtypeStruct((batch, emb_dim + _NOISE_DIM),
                                     jnp.float32),
  )(embs_pad, noise, labels[:, None], tail)
  return out


# speedup vs baseline: 1.0045x; 1.0045x over previous
import functools

import jax
import jax.numpy as jnp
from jax import lax
from jax.experimental import pallas as pl
from jax.experimental.pallas import tpu as pltpu
from jax.experimental.pallas import tpu_sc as plsc

_EMB_DIM = 64
_NOISE_DIM = 32
_NUM_CORES = 2
_NUM_SUBCORES = 16
_NW = _NUM_CORES * _NUM_SUBCORES

_N_ROWS = 1_000_000
_SWEEP_ROWS = (_N_ROWS // 128) * 128
_CHUNK = 512
_N_CHUNKS = _SWEEP_ROWS // _CHUNK
_CPW = _N_CHUNKS // _NW
_EXTRA = _N_CHUNKS - _CPW * _NW
_CAP = 1040
_SENT = 0x7FFFFF00


def _make_sweep_gather(batch: int):
  mesh = plsc.VectorSubcoreMesh(
      core_axis_name="c", subcore_axis_name="s",
      num_cores=_NUM_CORES, num_subcores=_NUM_SUBCORES)
  n_groups = batch // 16

  @functools.partial(
      pl.kernel,
      out_type=jax.ShapeDtypeStruct((batch, 128), jnp.float32),
      mesh=mesh,
      scratch_types=[
          pltpu.VMEM((batch + 16,), jnp.int32),
          pltpu.VMEM((_CAP + 16,), jnp.int32),
          pltpu.VMEM((_CAP + 16,), jnp.int32),
          pltpu.VMEM((_CAP + 16,), jnp.int32),
          pltpu.VMEM((_CAP + 16,), jnp.int32),
          pltpu.VMEM((2, _EMB_DIM, _CHUNK), jnp.float32),
          pltpu.VMEM((2, 16, 128), jnp.float32),
          pltpu.SemaphoreType.DMA((2, 2)),
          pltpu.SemaphoreType.DMA,
      ],
      compiler_params=pltpu.CompilerParams(needs_layout_passes=False),
  )
  def sweep_kernel(labels_hbm, tabT_hbm, out_hbm, lab_v, mylab_v, myj_v,
                   clab_v, cj_v, buf_v, row_v, sems, ssem):
    wid = lax.axis_index("s") * _NUM_CORES + lax.axis_index("c")
    n_my = _CPW + jnp.where(wid == 0, _EXTRA, 0)
    c0 = wid * _CPW + jnp.where(wid > 0, _EXTRA, 0)
    lo = c0 * _CHUNK
    hi = (c0 + n_my) * _CHUNK

    _H = _CHUNK // 2

    def _issue(t, b):
      s = pl.multiple_of((c0 + t) * _CHUNK, _CHUNK)
      for h in range(2):
        pltpu.async_copy(
            tabT_hbm.at[pl.ds(0, _EMB_DIM), pl.ds(s + h * _H, _H)],
            buf_v.at[b, pl.ds(0, _EMB_DIM), pl.ds(h * _H, _H)],
            sems.at[b, h])

    def _wait(b):
      for h in range(2):
        pltpu.make_async_copy(
            tabT_hbm.at[pl.ds(0, _EMB_DIM), pl.ds(0, _H)],
            buf_v.at[b, pl.ds(0, _EMB_DIM), pl.ds(h * _H, _H)],
            sems.at[b, h]).wait()

    _issue(0, 0)
    _issue(1, 1)

    pltpu.sync_copy(labels_hbm, lab_v.at[pl.ds(0, batch)])

    iota = lax.iota(jnp.int32, 16)

    @pl.loop(0, (_CAP + 16) // 16)
    def _fill(g):
      mylab_v[pl.ds(g * 16, 16)] = jnp.full((16,), _SENT, jnp.int32)

    @pl.loop(0, n_groups, init_carry=jnp.int32(0))
    def _compact(g, cur):
      v = lab_v[pl.ds(g * 16, 16)]
      jv = g * 16 + iota
      m = (v >= lo) & (v < hi)
      plsc.store_compressed(mylab_v.at[pl.ds(cur, 16)], v, mask=m)
      plsc.store_compressed(myj_v.at[pl.ds(cur, 16)], jv, mask=m)
      return cur + plsc.all_reduce_population_count(m)[0]

    cnt = _compact
    ng1 = lax.div(cnt + 15, jnp.int32(16))

    def _process(t, b):
      s = (c0 + t) * _CHUNK

      @pl.loop(0, (_CAP + 16) // 16)
      def _cfill(g):
        sl = pl.ds(g * 16, 16)
        clab_v[sl] = jnp.zeros((16,), jnp.int32)
        cj_v[sl] = jnp.full((16,), -1, jnp.int32)

      @pl.loop(0, ng1, init_carry=jnp.int32(0))
      def _cc(g, cur):
        v = mylab_v[pl.ds(g * 16, 16)]
        jv = myj_v[pl.ds(g * 16, 16)]
        m = (v >= s) & (v < s + _CHUNK)
        plsc.store_compressed(clab_v.at[pl.ds(cur, 16)], v - s, mask=m)
        plsc.store_compressed(cj_v.at[pl.ds(cur, 16)], jv, mask=m)
        return cur + plsc.all_reduce_population_count(m)[0]

      cnt2 = _cc
      nw = lax.div(cnt2 + 15, jnp.int32(16))

      @pl.loop(0, nw)
      def _wave(wv):
        r = wv & 1

        @pl.when(wv >= 2)
        def _():
          pltpu.make_async_copy(
              row_v.at[0],
              out_hbm.at[plsc.Indices(cj_v.at[pl.ds(0, 16)],
                                      ignored_value=-1)],
              ssem).wait()

        cols = clab_v[pl.ds(wv * 16, 16)]
        for l in range(16):
          col = cols[l]
          for k in range(_EMB_DIM // 16):
            vals = plsc.load_gather(
                buf_v.at[b],
                [k * 16 + iota, jnp.full((16,), col, jnp.int32)])
            row_v[r, l, pl.ds(k * 16, 16)] = vals

        pltpu.async_copy(
            row_v.at[r],
            out_hbm.at[plsc.Indices(cj_v.at[pl.ds(wv * 16, 16)],
                                    ignored_value=-1)],
            ssem)

      @pl.when(nw >= 1)
      def _():
        pltpu.make_async_copy(
            row_v.at[0],
            out_hbm.at[plsc.Indices(cj_v.at[pl.ds(0, 16)],
                                    ignored_value=-1)],
            ssem).wait()

      @pl.when(nw >= 2)
      def _():
        pltpu.make_async_copy(
            row_v.at[0],
            out_hbm.at[plsc.Indices(cj_v.at[pl.ds(0, 16)],
                                    ignored_value=-1)],
            ssem).wait()

    @pl.loop(0, (_CPW - 1) // 2)
    def _sweep(q):
      t0 = q * 2
      _wait(0)
      _process(t0, 0)
      _issue(t0 + 2, 0)
      _wait(1)
      _process(t0 + 1, 1)
      @pl.when(t0 + 3 <= _CPW - 1)
      def _():
        _issue(t0 + 3, 1)

    _wait(0)
    _process(_CPW - 1, 0)

    @pl.when(wid == 0)
    def _():
      _issue(_CPW, 1)
      _wait(1)
      _process(_CPW, 1)

  return sweep_kernel


def _post_kernel(embs_ref, noise_ref, lab_ref, tail_ref, out_ref, *, emb_dim):
  e = embs_ref[...][:, :emb_dim]
  lab = lab_ref[...]
  tail_id = lab - _SWEEP_ROWS
  onehot = (tail_id == jax.lax.broadcasted_iota(jnp.int32, (1, _N_ROWS - _SWEEP_ROWS), 1)).astype(jnp.float32)
  tail_e = jnp.dot(onehot, tail_ref[...], preferred_element_type=jnp.float32)
  e = jnp.where(tail_id >= 0, tail_e, e)
  mean = jnp.mean(e, axis=-1, keepdims=True)
  var = jnp.sum((e - mean) ** 2, axis=-1, keepdims=True) / (emb_dim - 1)
  std = jnp.sqrt(var)
  z = std * noise_ref[...] + mean
  out_ref[...] = jnp.concatenate((z, e), axis=-1)


def kernel(labels, table):
  batch = labels.shape[0]
  n_rows, emb_dim = table.shape
  labels = labels.astype(jnp.int32)
  noise = jax.random.normal(jax.random.key(42), (batch, _NOISE_DIM),
                            dtype=jnp.float32)

  embs_pad = _make_sweep_gather(batch)(labels, table.T)

  tail = lax.slice(table, (_SWEEP_ROWS, 0), (n_rows, emb_dim))

  blk = 4096
  out = pl.pallas_call(
      functools.partial(_post_kernel, emb_dim=emb_dim),
      grid=(batch // blk,),
      in_specs=[
          pl.BlockSpec((blk, 128), lambda i: (i, 0)),
          pl.BlockSpec((blk, _NOISE_DIM), lambda i: (i, 0)),
          pl.BlockSpec((blk, 1), lambda i: (i, 0)),
          pl.BlockSpec((n_rows - _SWEEP_ROWS, emb_dim), lambda i: (0, 0)),
      ],
      out_specs=pl.BlockSpec((blk, emb_dim + _NOISE_DIM), lambda i: (i, 0)),
      out_shape=jax.ShapeDtypeStruct((batch, emb_dim + _NOISE_DIM),
                                     jnp.float32),
  )(embs_pad, noise, labels[:, None], tail)
  return out


# speedup vs baseline: 1.0464x; 1.0417x over previous
import functools

import jax
import jax.numpy as jnp
from jax import lax
from jax.experimental import pallas as pl
from jax.experimental.pallas import tpu as pltpu
from jax.experimental.pallas import tpu_sc as plsc

_EMB_DIM = 64
_NOISE_DIM = 32
_NUM_CORES = 2
_NUM_SUBCORES = 16
_NW = _NUM_CORES * _NUM_SUBCORES

_N_ROWS = 1_000_000
_SWEEP_ROWS = (_N_ROWS // 128) * 128
_CHUNK = 512
_N_CHUNKS = _SWEEP_ROWS // _CHUNK
_CPW = _N_CHUNKS // _NW
_EXTRA = _N_CHUNKS - _CPW * _NW
_CAP = 1040
_SENT = 0x7FFFFF00


def _make_sweep_gather(batch: int):
  mesh = plsc.VectorSubcoreMesh(
      core_axis_name="c", subcore_axis_name="s",
      num_cores=_NUM_CORES, num_subcores=_NUM_SUBCORES)
  n_groups = batch // 16

  @functools.partial(
      pl.kernel,
      out_type=jax.ShapeDtypeStruct((batch, 128), jnp.float32),
      mesh=mesh,
      scratch_types=[
          pltpu.VMEM((batch + 16,), jnp.int32),
          pltpu.VMEM((_CAP + 16,), jnp.int32),
          pltpu.VMEM((_CAP + 16,), jnp.int32),
          pltpu.VMEM((_CAP + 16,), jnp.int32),
          pltpu.VMEM((_CAP + 16,), jnp.int32),
          pltpu.VMEM((2, _EMB_DIM, _CHUNK), jnp.float32),
          pltpu.VMEM((2, 16, 128), jnp.float32),
          pltpu.SemaphoreType.DMA((2, 2)),
          pltpu.SemaphoreType.DMA,
      ],
      compiler_params=pltpu.CompilerParams(needs_layout_passes=False),
  )
  def sweep_kernel(labels_hbm, tabT_hbm, out_hbm, lab_v, mylab_v, myj_v,
                   clab_v, cj_v, buf_v, row_v, sems, ssem):
    wid = lax.axis_index("s") * _NUM_CORES + lax.axis_index("c")
    n_my = _CPW + jnp.where(wid == 0, _EXTRA, 0)
    c0 = wid * _CPW + jnp.where(wid > 0, _EXTRA, 0)
    lo = c0 * _CHUNK
    hi = (c0 + n_my) * _CHUNK

    _H = _CHUNK // 2

    def _issue(t, b):
      s = pl.multiple_of((c0 + t) * _CHUNK, _CHUNK)
      for h in range(2):
        pltpu.async_copy(
            tabT_hbm.at[pl.ds(0, _EMB_DIM), pl.ds(s + h * _H, _H)],
            buf_v.at[b, pl.ds(0, _EMB_DIM), pl.ds(h * _H, _H)],
            sems.at[b, h])

    def _wait(b):
      for h in range(2):
        pltpu.make_async_copy(
            tabT_hbm.at[pl.ds(0, _EMB_DIM), pl.ds(0, _H)],
            buf_v.at[b, pl.ds(0, _EMB_DIM), pl.ds(h * _H, _H)],
            sems.at[b, h]).wait()

    _issue(0, 0)
    _issue(1, 1)

    pltpu.sync_copy(labels_hbm, lab_v.at[pl.ds(0, batch)])

    iota = lax.iota(jnp.int32, 16)

    @pl.loop(0, (_CAP + 16) // 16)
    def _fill(g):
      mylab_v[pl.ds(g * 16, 16)] = jnp.full((16,), _SENT, jnp.int32)

    @pl.loop(0, n_groups, init_carry=jnp.int32(0))
    def _compact(g, cur):
      v = lab_v[pl.ds(g * 16, 16)]
      jv = g * 16 + iota
      m = (v >= lo) & (v < hi)
      plsc.store_compressed(mylab_v.at[pl.ds(cur, 16)], v, mask=m)
      plsc.store_compressed(myj_v.at[pl.ds(cur, 16)], jv, mask=m)
      return cur + plsc.all_reduce_population_count(m)[0]

    cnt = _compact
    ng1 = lax.div(cnt + 15, jnp.int32(16))

    def _process(t, b):
      s = (c0 + t) * _CHUNK

      @pl.loop(0, (_CAP + 16) // 16)
      def _cfill(g):
        sl = pl.ds(g * 16, 16)
        clab_v[sl] = jnp.zeros((16,), jnp.int32)
        cj_v[sl] = jnp.full((16,), -1, jnp.int32)

      @pl.loop(0, ng1, init_carry=jnp.int32(0))
      def _cc(g, cur):
        v = mylab_v[pl.ds(g * 16, 16)]
        jv = myj_v[pl.ds(g * 16, 16)]
        m = (v >= s) & (v < s + _CHUNK)
        plsc.store_compressed(clab_v.at[pl.ds(cur, 16)], v - s, mask=m)
        plsc.store_compressed(cj_v.at[pl.ds(cur, 16)], jv, mask=m)
        return cur + plsc.all_reduce_population_count(m)[0]

      cnt2 = _cc
      nw = lax.div(cnt2 + 15, jnp.int32(16))

      @pl.loop(0, nw)
      def _wave(wv):
        r = wv & 1

        @pl.when(wv >= 2)
        def _():
          pltpu.make_async_copy(
              row_v.at[0],
              out_hbm.at[plsc.Indices(cj_v.at[pl.ds(0, 16)],
                                      ignored_value=-1)],
              ssem).wait()

        cols = clab_v[pl.ds(wv * 16, 16)]
        for l in range(16):
          col = cols[l]
          for k in range(_EMB_DIM // 16):
            vals = plsc.load_gather(
                buf_v.at[b],
                [k * 16 + iota, jnp.full((16,), col, jnp.int32)])
            row_v[r, l, pl.ds(k * 16, 16)] = vals

        pltpu.async_copy(
            row_v.at[r],
            out_hbm.at[plsc.Indices(cj_v.at[pl.ds(wv * 16, 16)],
                                    ignored_value=-1)],
            ssem)

      @pl.when(nw >= 1)
      def _():
        pltpu.make_async_copy(
            row_v.at[0],
            out_hbm.at[plsc.Indices(cj_v.at[pl.ds(0, 16)],
                                    ignored_value=-1)],
            ssem).wait()

      @pl.when(nw >= 2)
      def _():
        pltpu.make_async_copy(
            row_v.at[0],
            out_hbm.at[plsc.Indices(cj_v.at[pl.ds(0, 16)],
                                    ignored_value=-1)],
            ssem).wait()

    @pl.loop(0, (_CPW - 1) // 2)
    def _sweep(q):
      t0 = q * 2
      _wait(0)
      _process(t0, 0)
      _issue(t0 + 2, 0)
      _wait(1)
      _process(t0 + 1, 1)
      @pl.when(t0 + 3 <= _CPW - 1)
      def _():
        _issue(t0 + 3, 1)

    _wait(0)
    _process(_CPW - 1, 0)

    @pl.when(wid == 0)
    def _():
      _issue(_CPW, 1)
      _wait(1)
      _process(_CPW, 1)

  return sweep_kernel


def _post_kernel(embs_ref, noise_ref, lab_ref, tail_ref, out_ref, *, emb_dim):
  e = embs_ref[...][:, :emb_dim]
  lab = lab_ref[...]
  tail_id = lab - _SWEEP_ROWS
  onehot = (tail_id == jax.lax.broadcasted_iota(jnp.int32, (1, _N_ROWS - _SWEEP_ROWS), 1)).astype(jnp.float32)
  tail_e = jnp.dot(onehot, tail_ref[...], preferred_element_type=jnp.float32)
  e = jnp.where(tail_id >= 0, tail_e, e)
  mean = jnp.mean(e, axis=-1, keepdims=True)
  var = jnp.sum((e - mean) ** 2, axis=-1, keepdims=True) / (emb_dim - 1)
  std = jnp.sqrt(var)
  z = std * noise_ref[...] + mean
  out_ref[...] = jnp.concatenate((z, e), axis=-1).T


def kernel(labels, table):
  batch = labels.shape[0]
  n_rows, emb_dim = table.shape
  labels = labels.astype(jnp.int32)
  noise = jax.random.normal(jax.random.key(42), (batch, _NOISE_DIM),
                            dtype=jnp.float32)

  embs_pad = _make_sweep_gather(batch)(labels, table.T)

  tail = lax.slice(table, (_SWEEP_ROWS, 0), (n_rows, emb_dim))

  blk = 4096
  out = pl.pallas_call(
      functools.partial(_post_kernel, emb_dim=emb_dim),
      grid=(batch // blk,),
      in_specs=[
          pl.BlockSpec((blk, 128), lambda i: (i, 0)),
          pl.BlockSpec((blk, _NOISE_DIM), lambda i: (i, 0)),
          pl.BlockSpec((blk, 1), lambda i: (i, 0)),
          pl.BlockSpec((n_rows - _SWEEP_ROWS, emb_dim), lambda i: (0, 0)),
      ],
      out_specs=pl.BlockSpec((emb_dim + _NOISE_DIM, blk), lambda i: (0, i)),
      out_shape=jax.ShapeDtypeStruct((emb_dim + _NOISE_DIM, batch),
                                     jnp.float32),
  )(embs_pad, noise, labels[:, None], tail)
  return out.T
